# baseline (speedup 1.0000x reference)
import functools

import jax
import jax.numpy as jnp
from jax import lax
from jax.experimental import pallas as pl
from jax.experimental.pallas import tpu as pltpu
from jax.experimental.pallas import tpu_sc as plsc

DIM = 32
IT = 0.01
SC_OFF = 2 * IT * 0.5772156649015329
LN2 = 0.6931471805599453
SQRT2 = 1.4142135623730951


def _vlog(x):
    xi = lax.bitcast_convert_type(x, jnp.int32)
    e = lax.shift_right_arithmetic(xi, 23) - 127
    m = lax.bitcast_convert_type(
        jnp.bitwise_or(jnp.bitwise_and(xi, 0x7FFFFF), 0x3F800000), jnp.float32)
    big = m > SQRT2
    m = jnp.where(big, m * 0.5, m)
    e = jnp.where(big, e + 1, e)
    s = (m - 1.0) / (m + 1.0)
    z = s * s
    p = 2.0 * s * (1.0 + z * (1.0 / 3.0 + z * (0.2 + z * (1.0 / 7.0 + z / 9.0))))
    return p + e.astype(jnp.float32) * LN2


def _log1p(t):
    return _vlog(1.0 + t)


def _softplus(x):
    return jnp.maximum(x, 0.0) + _log1p(jnp.exp(-jnp.abs(x)))


def _logaddexp(a, b):
    return jnp.maximum(a, b) + _log1p(jnp.exp(-jnp.abs(a - b)))


def _term(c1, e1b, b1, c2, e2b, b2):
    w1 = _softplus(e1b) * 0.5
    w2 = _softplus(e2b) * 0.5
    min1 = c1 - w1
    max1 = c1 + w1
    min2 = c2 - w2
    max2 = c2 + w2
    bin_vec = 1.0 / ((1.0 + jnp.exp(-b1)) * (1.0 + jnp.exp(-b2)))
    meet_min = IT * _logaddexp(min1 / IT, min2 / IT)
    meet_max = -IT * _logaddexp(-max1 / IT, -max2 / IT)
    meet_min = jnp.maximum(meet_min, jnp.maximum(min1, min2))
    meet_max = jnp.minimum(meet_max, jnp.minimum(max1, max2))
    lv_meet = _vlog(_softplus(meet_max - meet_min - SC_OFF) + 1e-20)
    lv_rhs = _vlog(_softplus(max2 - min2 - SC_OFF) + 1e-20)
    return (lv_meet - lv_rhs) * bin_vec


def _make_sc_kernel(B, N):
    info = plsc.get_sparse_core_info()
    NC, NS = info.num_cores, info.num_subcores
    NW = NC * NS
    b_per_w = B // NW
    CH = 256
    n_ch = b_per_w // CH
    NG = CH // 16

    @functools.partial(
        pl.kernel,
        mesh=plsc.VectorSubcoreMesh(core_axis_name="c", subcore_axis_name="s"),
        compiler_params=pltpu.CompilerParams(use_tc_tiling_on_sc=False),
        out_type=jax.ShapeDtypeStruct((B,), jnp.float32),
        scratch_types=[
            pltpu.VMEM((CH,), jnp.int32),
            pltpu.VMEM((CH,), jnp.int32),
            pltpu.VMEM((DIM * CH,), jnp.int32),
            pltpu.VMEM((DIM * CH,), jnp.int32),
            pltpu.VMEM((DIM * CH,), jnp.float32),
            pltpu.VMEM((DIM * CH,), jnp.float32),
            pltpu.VMEM((DIM * CH,), jnp.float32),
            pltpu.VMEM((DIM * CH,), jnp.float32),
            pltpu.VMEM((DIM * CH,), jnp.float32),
            pltpu.VMEM((DIM * CH,), jnp.float32),
            pltpu.VMEM((CH,), jnp.float32),
            pltpu.SemaphoreType.DMA,
        ],
    )
    def sc_k(idx1_hbm, idx2_hbm, t1, t2, t3, out_hbm,
             i1_v, i2_v, ib1, ib2, d0, d1, d2, d3, d4, d5, ob, sem):
        wid = lax.axis_index("s") * NC + lax.axis_index("c")
        base = wid * b_per_w

        for ch in range(n_ch):
            off = base + ch * CH
            pltpu.sync_copy(idx1_hbm.at[pl.ds(off, CH)], i1_v)
            pltpu.sync_copy(idx2_hbm.at[pl.ds(off, CH)], i2_v)

            def build(j, _):
                v1 = i1_v[pl.ds(j * 16, 16)]
                v2 = i2_v[pl.ds(j * 16, 16)]

                def per_d(d, _):
                    dn = d * N
                    ib1[pl.ds(d * CH + j * 16, 16)] = v1 + dn
                    ib2[pl.ds(d * CH + j * 16, 16)] = v2 + dn
                    return 0

                return lax.fori_loop(0, DIM, per_d, 0)

            lax.fori_loop(0, NG, build, 0)

            cps = [pltpu.async_copy(t1.at[ib1], d0, sem),
                   pltpu.async_copy(t2.at[ib1], d1, sem),
                   pltpu.async_copy(t3.at[ib1], d2, sem),
                   pltpu.async_copy(t1.at[ib2], d3, sem),
                   pltpu.async_copy(t2.at[ib2], d4, sem),
                   pltpu.async_copy(t3.at[ib2], d5, sem)]
            for cp in cps:
                cp.wait()

            def group(pg, _):
                def per_d(d, acc):
                    q = d * CH + pg * 16
                    return acc + _term(d0[pl.ds(q, 16)], d1[pl.ds(q, 16)],
                                       d2[pl.ds(q, 16)], d3[pl.ds(q, 16)],
                                       d4[pl.ds(q, 16)], d5[pl.ds(q, 16)])

                acc = lax.fori_loop(0, DIM, per_d, jnp.zeros(16, jnp.float32))
                ob[pl.ds(pg * 16, 16)] = acc
                return 0

            lax.fori_loop(0, NG, group, 0)
            pltpu.sync_copy(ob, out_hbm.at[pl.ds(off, CH)])

    return sc_k


def kernel(idx1, idx2, emb1, emb2, embs1_w, embs2_w, bins_w):
    del emb1, emb2
    B = idx1.shape[0]
    N = embs1_w.shape[0]
    t1 = jnp.transpose(embs1_w).reshape(-1)
    t2 = jnp.transpose(embs2_w).reshape(-1)
    t3 = jnp.transpose(bins_w).reshape(-1)
    return _make_sc_kernel(B, N)(idx1, idx2, t1, t2, t3)


# speedup vs baseline: 5.7064x; 5.7064x over previous
import functools

import jax
import jax.numpy as jnp
from jax import lax
from jax.experimental import pallas as pl
from jax.experimental.pallas import tpu as pltpu
from jax.experimental.pallas import tpu_sc as plsc

DIM = 32
VT = 1.0
IT = 0.01
SC_OFF = 2 * IT * 0.5772156649015329


def _gather_stage(idx1, idx2, embs1_w, embs2_w, bins_w):
    B = idx1.shape[0]
    info = plsc.get_sparse_core_info()
    NC, NS = info.num_cores, info.num_subcores
    NW = NC * NS
    b_per_w = B // NW

    @functools.partial(
        pl.kernel,
        mesh=plsc.VectorSubcoreMesh(core_axis_name="c", subcore_axis_name="s"),
        compiler_params=pltpu.CompilerParams(use_tc_tiling_on_sc=False),
        out_type=jax.ShapeDtypeStruct((6, B, DIM), jnp.float32),
        scratch_types=[
            pltpu.VMEM((b_per_w,), jnp.int32),
            pltpu.VMEM((b_per_w,), jnp.int32),
            pltpu.VMEM((b_per_w, DIM), jnp.float32),
            pltpu.VMEM((b_per_w, DIM), jnp.float32),
            pltpu.VMEM((b_per_w, DIM), jnp.float32),
            pltpu.VMEM((b_per_w, DIM), jnp.float32),
            pltpu.VMEM((b_per_w, DIM), jnp.float32),
            pltpu.VMEM((b_per_w, DIM), jnp.float32),
            pltpu.SemaphoreType.DMA,
        ],
    )
    def gather_k(idx1_hbm, idx2_hbm, t1, t2, t3, out_hbm,
                 i1_v, i2_v, r0, r1, r2, r3, r4, r5, sem):
        wid = lax.axis_index("s") * NC + lax.axis_index("c")
        base = wid * b_per_w
        pltpu.sync_copy(idx1_hbm.at[pl.ds(base, b_per_w)], i1_v)
        pltpu.sync_copy(idx2_hbm.at[pl.ds(base, b_per_w)], i2_v)
        rows = (r0, r1, r2, r3, r4, r5)
        plan = ((t1, i1_v), (t2, i1_v), (t3, i1_v),
                (t1, i2_v), (t2, i2_v), (t3, i2_v))
        copies = [pltpu.async_copy(table.at[iv], rows[t], sem)
                  for t, (table, iv) in enumerate(plan)]
        for c in copies:
            c.wait()
        for t in range(6):
            pltpu.sync_copy(rows[t], out_hbm.at[t, pl.ds(base, b_per_w)])

    return gather_k(idx1, idx2, embs1_w, embs2_w, bins_w)


def _log_volume(z, Z, c):
    return jnp.sum(jnp.log(VT * jax.nn.softplus((Z - z - SC_OFF) / VT) + 1e-20) * c,
                   axis=-1)


def _compute_body(g_ref, o_ref):
    sp = jax.nn.softplus
    c1 = g_ref[0]
    w1 = sp(g_ref[1]) / 2
    bin1 = g_ref[2]
    c2 = g_ref[3]
    w2 = sp(g_ref[4]) / 2
    bin2 = g_ref[5]
    min1 = c1 - w1
    max1 = c1 + w1
    min2 = c2 - w2
    max2 = c2 + w2
    bin_vec = jax.nn.sigmoid(bin1) * jax.nn.sigmoid(bin2)
    meet_min = IT * jnp.logaddexp(min1 / IT, min2 / IT)
    meet_max = -IT * jnp.logaddexp(-max1 / IT, -max2 / IT)
    meet_min = jnp.maximum(meet_min, jnp.maximum(min1, min2))
    meet_max = jnp.minimum(meet_max, jnp.minimum(max1, max2))
    logp = _log_volume(meet_min, meet_max, bin_vec) - _log_volume(min2, max2, bin_vec)
    o_ref[...] = logp


def _compute_stage(g):
    B = g.shape[1]
    bs = 2048
    grid = B // bs
    out = pl.pallas_call(
        _compute_body,
        grid=(grid,),
        in_specs=[pl.BlockSpec((6, bs, DIM), lambda i: (0, i, 0))],
        out_specs=pl.BlockSpec((bs,), lambda i: (i,)),
        out_shape=jax.ShapeDtypeStruct((B,), jnp.float32),
    )(g)
    return out


def kernel(idx1, idx2, emb1, emb2, embs1_w, embs2_w, bins_w):
    del emb1, emb2
    g = _gather_stage(idx1, idx2, embs1_w, embs2_w, bins_w)
    return _compute_stage(g)
